# Initial kernel scaffold; baseline (speedup 1.0000x reference)
import functools

import jax
import jax.numpy as jnp
from jax import lax
from jax.experimental import pallas as pl
from jax.experimental.pallas import tpu as pltpu
from jax.experimental.pallas import tpu_sc as plsc

NINP = 128
D1 = 32
D2 = 8
N_TOK = 16384
C1 = 20000
C2 = 200000
C3 = 1000000
NHEAD = C1
NT0 = C2 - C1
NT1 = C3 - C2

NC = 2
NS = 16
NW = NC * NS
BPW = N_TOK // NW
L = 16
GCH = 128
NCH = BPW // GCH


def _stage_a_body(ids_hbm, head_hbm, t0_hbm, t1_hbm,
                  gh_out, g0_out, g1_out,
                  ids_v, hidx_v, i0_v, i1_v, gh_v, g0_v, g1_v, sem):
    wid = lax.axis_index("s") * NC + lax.axis_index("c")
    base = wid * BPW

    pltpu.sync_copy(ids_hbm.at[pl.ds(base, BPW)], ids_v)

    for i in range(BPW // L):
        v = ids_v[pl.ds(i * L, L)]
        hidx = jnp.minimum(v, NHEAD - 1)
        i0 = jnp.clip(v - C1, 0, NT0 - 1)
        i1 = jnp.clip(v - C2, 0, NT1 - 1)
        r, c = i // (GCH // L), (i % (GCH // L)) * L
        hidx_v[r, pl.ds(c, L)] = hidx
        i0_v[r, pl.ds(c, L)] = i0
        i1_v[r, pl.ds(c, L)] = i1

    copies = []
    for ch in range(NCH):
        sl = pl.ds(ch * GCH, GCH)
        copies.append(pltpu.async_copy(head_hbm.at[hidx_v.at[ch]], gh_v.at[sl], sem))
        copies.append(pltpu.async_copy(t0_hbm.at[i0_v.at[ch]], g0_v.at[sl], sem))
        copies.append(pltpu.async_copy(t1_hbm.at[i1_v.at[ch]], g1_v.at[sl], sem))
    for cp in copies:
        cp.wait()

    pltpu.sync_copy(gh_v, gh_out.at[pl.ds(base, BPW)])
    pltpu.sync_copy(g0_v, g0_out.at[pl.ds(base, BPW)])
    pltpu.sync_copy(g1_v, g1_out.at[pl.ds(base, BPW)])


_stage_a = functools.partial(
    pl.kernel,
    mesh=plsc.VectorSubcoreMesh(core_axis_name="c", subcore_axis_name="s"),
    out_type=[
        jax.ShapeDtypeStruct((N_TOK, NINP), jnp.float32),
        jax.ShapeDtypeStruct((N_TOK, D1), jnp.float32),
        jax.ShapeDtypeStruct((N_TOK, D2), jnp.float32),
    ],
    scratch_types=[
        pltpu.VMEM((BPW,), jnp.int32),
        pltpu.VMEM((NCH, GCH), jnp.int32),
        pltpu.VMEM((NCH, GCH), jnp.int32),
        pltpu.VMEM((NCH, GCH), jnp.int32),
        pltpu.VMEM((BPW, NINP), jnp.float32),
        pltpu.VMEM((BPW, D1), jnp.float32),
        pltpu.VMEM((BPW, D2), jnp.float32),
        pltpu.SemaphoreType.DMA,
    ],
)(_stage_a_body)


TB = 1024


def _stage_b_body(ids_ref, gh_ref, g0_ref, g1_ref, p0t_ref, p1t_ref, out_ref):
    ids = ids_ref[...]
    y0 = jnp.dot(g0_ref[...], p0t_ref[...], preferred_element_type=jnp.float32)
    y1 = jnp.dot(g1_ref[...], p1t_ref[...], preferred_element_type=jnp.float32)
    m0 = ids < C1
    m1 = ids < C2
    out_ref[...] = jnp.where(m0, gh_ref[...], jnp.where(m1, y0, y1))


def kernel(input, head_w, tail0_emb, tail0_proj, tail1_emb, tail1_proj):
    gh, g0, g1 = _stage_a(input, head_w, tail0_emb, tail1_emb)
    ids2d = input.reshape(N_TOK, 1)
    p0t = tail0_proj.T
    p1t = tail1_proj.T
    out = pl.pallas_call(
        _stage_b_body,
        grid=(N_TOK // TB,),
        in_specs=[
            pl.BlockSpec((TB, 1), lambda i: (i, 0)),
            pl.BlockSpec((TB, NINP), lambda i: (i, 0)),
            pl.BlockSpec((TB, D1), lambda i: (i, 0)),
            pl.BlockSpec((TB, D2), lambda i: (i, 0)),
            pl.BlockSpec((D1, NINP), lambda i: (0, 0)),
            pl.BlockSpec((D2, NINP), lambda i: (0, 0)),
        ],
        out_specs=pl.BlockSpec((TB, NINP), lambda i: (i, 0)),
        out_shape=jax.ShapeDtypeStruct((N_TOK, NINP), jnp.float32),
    )(ids2d, gh, g0, g1, p0t, p1t)
    return out


# speedup vs baseline: 2.6158x; 2.6158x over previous
import functools

import jax
import jax.numpy as jnp
from jax import lax
from jax.experimental import pallas as pl
from jax.experimental.pallas import tpu as pltpu
from jax.experimental.pallas import tpu_sc as plsc

NINP = 128
D1 = 32
D2 = 8
N_TOK = 16384
C1 = 20000
C2 = 200000
C3 = 1000000
NHEAD = C1
NT0 = C2 - C1
NT1 = C3 - C2

NC = 2
NS = 16
NW = NC * NS
BPW = N_TOK // NW
L = 16
GCH = 128
NCH = BPW // GCH
DUMMY_MASK = 0x3FFF


def _stage_a_body(ids_hbm, head_hbm, t0_hbm, t1_hbm,
                  gh_out, g0_out, g1_out,
                  ids_v, hidx_v, i0_v, i1_v, gh_v, g0_v, g1_v, sem):
    wid = lax.axis_index("s") * NC + lax.axis_index("c")
    base = wid * BPW

    pltpu.sync_copy(ids_hbm.at[pl.ds(base, BPW)], ids_v)

    for i in range(BPW // L):
        v = ids_v[pl.ds(i * L, L)]
        spread = v & DUMMY_MASK
        hidx = jnp.where(v < C1, v, spread)
        in0 = (v >= C1) & (v < C2)
        i0 = jnp.where(in0, v - C1, spread)
        in1 = v >= C2
        i1 = jnp.where(in1, v - C2, spread)
        r, c = i // (GCH // L), (i % (GCH // L)) * L
        hidx_v[r, pl.ds(c, L)] = hidx
        i0_v[r, pl.ds(c, L)] = i0
        i1_v[r, pl.ds(c, L)] = i1

    copies = []
    for ch in range(NCH):
        sl = pl.ds(ch * GCH, GCH)
        copies.append(pltpu.async_copy(head_hbm.at[hidx_v.at[ch]], gh_v.at[sl], sem))
        copies.append(pltpu.async_copy(t0_hbm.at[i0_v.at[ch]], g0_v.at[sl], sem))
        copies.append(pltpu.async_copy(t1_hbm.at[i1_v.at[ch]], g1_v.at[sl], sem))
    for cp in copies:
        cp.wait()

    pltpu.sync_copy(gh_v, gh_out.at[pl.ds(base, BPW)])
    pltpu.sync_copy(g0_v, g0_out.at[pl.ds(base, BPW)])
    pltpu.sync_copy(g1_v, g1_out.at[pl.ds(base, BPW)])


_stage_a = functools.partial(
    pl.kernel,
    mesh=plsc.VectorSubcoreMesh(core_axis_name="c", subcore_axis_name="s"),
    compiler_params=pltpu.CompilerParams(use_tc_tiling_on_sc=False),
    out_type=[
        jax.ShapeDtypeStruct((N_TOK, NINP), jnp.float32),
        jax.ShapeDtypeStruct((N_TOK, D1), jnp.float32),
        jax.ShapeDtypeStruct((N_TOK, D2), jnp.float32),
    ],
    scratch_types=[
        pltpu.VMEM((BPW,), jnp.int32),
        pltpu.VMEM((NCH, GCH), jnp.int32),
        pltpu.VMEM((NCH, GCH), jnp.int32),
        pltpu.VMEM((NCH, GCH), jnp.int32),
        pltpu.VMEM((BPW, NINP), jnp.float32),
        pltpu.VMEM((BPW, D1), jnp.float32),
        pltpu.VMEM((BPW, D2), jnp.float32),
        pltpu.SemaphoreType.DMA,
    ],
)(_stage_a_body)


TB = 1024


def _stage_b_body(ids_ref, gh_ref, g0_ref, g1_ref, p0t_ref, p1t_ref, out_ref):
    ids = ids_ref[...]
    y0 = jnp.dot(g0_ref[...], p0t_ref[...], preferred_element_type=jnp.float32)
    y1 = jnp.dot(g1_ref[...], p1t_ref[...], preferred_element_type=jnp.float32)
    m0 = ids < C1
    m1 = ids < C2
    out_ref[...] = jnp.where(m0, gh_ref[...], jnp.where(m1, y0, y1))


def kernel(input, head_w, tail0_emb, tail0_proj, tail1_emb, tail1_proj):
    gh, g0, g1 = _stage_a(input, head_w, tail0_emb, tail1_emb)
    ids2d = input.reshape(N_TOK, 1)
    p0t = tail0_proj.T
    p1t = tail1_proj.T
    out = pl.pallas_call(
        _stage_b_body,
        grid=(N_TOK // TB,),
        in_specs=[
            pl.BlockSpec((TB, 1), lambda i: (i, 0)),
            pl.BlockSpec((TB, NINP), lambda i: (i, 0)),
            pl.BlockSpec((TB, D1), lambda i: (i, 0)),
            pl.BlockSpec((TB, D2), lambda i: (i, 0)),
            pl.BlockSpec((D1, NINP), lambda i: (0, 0)),
            pl.BlockSpec((D2, NINP), lambda i: (0, 0)),
        ],
        out_specs=pl.BlockSpec((TB, NINP), lambda i: (i, 0)),
        out_shape=jax.ShapeDtypeStruct((N_TOK, NINP), jnp.float32),
    )(ids2d, gh, g0, g1, p0t, p1t)
    return out


# speedup vs baseline: 7.2707x; 2.7795x over previous
import functools

import jax
import jax.numpy as jnp
from jax import lax
from jax.experimental import pallas as pl
from jax.experimental.pallas import tpu as pltpu
from jax.experimental.pallas import tpu_sc as plsc

NINP = 128
D1 = 32
D2 = 8
N_TOK = 16384
C1 = 20000
C2 = 200000
C3 = 1000000
NHEAD = C1
NT0 = C2 - C1
NT1 = C3 - C2

NC = 2
NS = 16
NW = NC * NS
BPW = N_TOK // NW
L = 16
GCH = 128
NCH = BPW // GCH
DUMMY_MASK = 0x3FFF


NE1 = BPW * D2
NCH1 = NE1 // GCH


def _stage_a_body(ids_hbm, head_hbm, t0_hbm, t1f_hbm,
                  gh_out, g0_out, g1f_out,
                  ids_v, hidx_v, i0_v, i1e_v, gh_v, g0_v, g1f_v, sem):
    wid = lax.axis_index("s") * NC + lax.axis_index("c")
    base = wid * BPW

    pltpu.sync_copy(ids_hbm.at[pl.ds(base, BPW)], ids_v)
    lanes = lax.iota(jnp.int32, L)

    for i in range(BPW // L):
        v = ids_v[pl.ds(i * L, L)]
        spread = v & DUMMY_MASK
        hidx = jnp.where(v < C1, v, spread)
        in0 = (v >= C1) & (v < C2)
        i0 = jnp.where(in0, v - C1, spread)
        r1 = jnp.where(v >= C2, v - C2, spread)
        e1 = ((r1 >> 7) << 10) + (r1 & 127)
        r, c = i // (GCH // L), (i % (GCH // L)) * L
        hidx_v[r, pl.ds(c, L)] = hidx
        i0_v[r, pl.ds(c, L)] = i0
        pdst = (lanes + i * L) * D2
        for k in range(D2):
            plsc.store_scatter(i1e_v, [pdst + k], e1 + (k << 7))

    copies = []
    for ch in range(NCH):
        sl = pl.ds(ch * GCH, GCH)
        copies.append(pltpu.async_copy(head_hbm.at[hidx_v.at[ch]], gh_v.at[sl], sem))
        copies.append(pltpu.async_copy(t0_hbm.at[i0_v.at[ch]], g0_v.at[sl], sem))
    for ch in range(NCH1):
        sl = pl.ds(ch * GCH, GCH)
        copies.append(pltpu.async_copy(t1f_hbm.at[i1e_v.at[sl]], g1f_v.at[sl], sem))
    for cp in copies:
        cp.wait()

    pltpu.sync_copy(gh_v, gh_out.at[pl.ds(base, BPW)])
    pltpu.sync_copy(g0_v, g0_out.at[pl.ds(base, BPW)])
    pltpu.sync_copy(g1f_v, g1f_out.at[pl.ds(base * D2, NE1)])


_stage_a = functools.partial(
    pl.kernel,
    mesh=plsc.VectorSubcoreMesh(core_axis_name="c", subcore_axis_name="s"),
    compiler_params=pltpu.CompilerParams(
        use_tc_tiling_on_sc=False, needs_layout_passes=False),
    out_type=[
        jax.ShapeDtypeStruct((N_TOK, NINP), jnp.float32),
        jax.ShapeDtypeStruct((N_TOK, D1), jnp.float32),
        jax.ShapeDtypeStruct((N_TOK * D2,), jnp.float32),
    ],
    scratch_types=[
        pltpu.VMEM((BPW,), jnp.int32),
        pltpu.VMEM((NCH, GCH), jnp.int32),
        pltpu.VMEM((NCH, GCH), jnp.int32),
        pltpu.VMEM((NE1,), jnp.int32),
        pltpu.VMEM((BPW, NINP), jnp.float32),
        pltpu.VMEM((BPW, D1), jnp.float32),
        pltpu.VMEM((NE1,), jnp.float32),
        pltpu.SemaphoreType.DMA,
    ],
)(_stage_a_body)


TB = 1024


def _stage_b_body(ids_ref, gh_ref, g0_ref, g1_ref, p0t_ref, p1t_ref, out_ref):
    ids = ids_ref[...]
    y0 = jnp.dot(g0_ref[...], p0t_ref[...], preferred_element_type=jnp.float32)
    y1 = jnp.dot(g1_ref[...], p1t_ref[...], preferred_element_type=jnp.float32)
    m0 = ids < C1
    m1 = ids < C2
    out_ref[...] = jnp.where(m0, gh_ref[...], jnp.where(m1, y0, y1))


def kernel(input, head_w, tail0_emb, tail0_proj, tail1_emb, tail1_proj):
    t1flat = tail1_emb.reshape(NT1 // 128, 128, D2).swapaxes(1, 2).reshape(-1)
    t0rm = jax.lax.optimization_barrier(tail0_emb.reshape(-1)).reshape(NT0, D1)
    gh, g0, g1f = _stage_a(input, head_w, t0rm, t1flat)
    g1 = g1f.reshape(N_TOK, D2)
    ids2d = input.reshape(N_TOK, 1)
    p0t = tail0_proj.T
    p1t = tail1_proj.T
    out = pl.pallas_call(
        _stage_b_body,
        grid=(N_TOK // TB,),
        in_specs=[
            pl.BlockSpec((TB, 1), lambda i: (i, 0)),
            pl.BlockSpec((TB, NINP), lambda i: (i, 0)),
            pl.BlockSpec((TB, D1), lambda i: (i, 0)),
            pl.BlockSpec((TB, D2), lambda i: (i, 0)),
            pl.BlockSpec((D1, NINP), lambda i: (0, 0)),
            pl.BlockSpec((D2, NINP), lambda i: (0, 0)),
        ],
        out_specs=pl.BlockSpec((TB, NINP), lambda i: (i, 0)),
        out_shape=jax.ShapeDtypeStruct((N_TOK, NINP), jnp.float32),
    )(ids2d, gh, g0, g1, p0t, p1t)
    return out


# speedup vs baseline: 9.8891x; 1.3601x over previous
import functools

import jax
import jax.numpy as jnp
from jax import lax
from jax.experimental.layout import Layout, with_layout_constraint
from jax.experimental import pallas as pl
from jax.experimental.pallas import tpu as pltpu
from jax.experimental.pallas import tpu_sc as plsc

NINP = 128
D1 = 32
D2 = 8
N_TOK = 16384
C1 = 20000
C2 = 200000
C3 = 1000000
NHEAD = C1
NT0 = C2 - C1
NT1 = C3 - C2

NC = 2
NS = 16
NW = NC * NS
BPW = N_TOK // NW
L = 16
GCH = 128
NCH = BPW // GCH
DUMMY_MASK = 0x3FFF


NE1 = BPW * D2
NCH1 = NE1 // GCH


def _stage_a_body(ids_hbm, head_hbm, t0_hbm, t1f_hbm,
                  gh_out, g0_out, g1f_out,
                  ids_v, hidx_v, i0_v, i1e_v, gh_v, g0_v, g1f_v, sem):
    wid = lax.axis_index("s") * NC + lax.axis_index("c")
    base = wid * BPW

    pltpu.sync_copy(ids_hbm.at[pl.ds(base, BPW)], ids_v)
    lanes = lax.iota(jnp.int32, L)

    for i in range(BPW // L):
        v = ids_v[pl.ds(i * L, L)]
        spread = v & DUMMY_MASK
        hidx = jnp.where(v < C1, v, spread)
        in0 = (v >= C1) & (v < C2)
        i0 = jnp.where(in0, v - C1, spread)
        r1 = jnp.where(v >= C2, v - C2, spread)
        e1 = ((r1 >> 7) << 10) + (r1 & 127)
        r, c = i // (GCH // L), (i % (GCH // L)) * L
        hidx_v[r, pl.ds(c, L)] = hidx
        i0_v[r, pl.ds(c, L)] = i0
        pdst = (lanes + i * L) * D2
        for k in range(D2):
            plsc.store_scatter(i1e_v, [pdst + k], e1 + (k << 7))

    copies = []
    for ch in range(NCH):
        sl = pl.ds(ch * GCH, GCH)
        copies.append(pltpu.async_copy(head_hbm.at[hidx_v.at[ch]], gh_v.at[sl], sem))
        copies.append(pltpu.async_copy(t0_hbm.at[i0_v.at[ch]], g0_v.at[sl], sem))
    for ch in range(NCH1):
        sl = pl.ds(ch * GCH, GCH)
        copies.append(pltpu.async_copy(t1f_hbm.at[i1e_v.at[sl]], g1f_v.at[sl], sem))
    for cp in copies:
        cp.wait()

    pltpu.sync_copy(gh_v, gh_out.at[pl.ds(base, BPW)])
    pltpu.sync_copy(g0_v, g0_out.at[pl.ds(base, BPW)])
    pltpu.sync_copy(g1f_v, g1f_out.at[pl.ds(base * D2, NE1)])


_stage_a = functools.partial(
    pl.kernel,
    mesh=plsc.VectorSubcoreMesh(core_axis_name="c", subcore_axis_name="s"),
    compiler_params=pltpu.CompilerParams(
        use_tc_tiling_on_sc=False, needs_layout_passes=False),
    out_type=[
        jax.ShapeDtypeStruct((N_TOK, NINP), jnp.float32),
        jax.ShapeDtypeStruct((N_TOK, D1), jnp.float32),
        jax.ShapeDtypeStruct((N_TOK * D2,), jnp.float32),
    ],
    scratch_types=[
        pltpu.VMEM((BPW,), jnp.int32),
        pltpu.VMEM((NCH, GCH), jnp.int32),
        pltpu.VMEM((NCH, GCH), jnp.int32),
        pltpu.VMEM((NE1,), jnp.int32),
        pltpu.VMEM((BPW, NINP), jnp.float32),
        pltpu.VMEM((BPW, D1), jnp.float32),
        pltpu.VMEM((NE1,), jnp.float32),
        pltpu.SemaphoreType.DMA,
    ],
)(_stage_a_body)


TB = 1024


def _stage_b_body(ids_ref, gh_ref, g0_ref, g1_ref, p0t_ref, p1t_ref, out_ref):
    ids = ids_ref[...]
    y0 = jnp.dot(g0_ref[...], p0t_ref[...], preferred_element_type=jnp.float32)
    y1 = jnp.dot(g1_ref[...], p1t_ref[...], preferred_element_type=jnp.float32)
    m0 = ids < C1
    m1 = ids < C2
    out_ref[...] = jnp.where(m0, gh_ref[...], jnp.where(m1, y0, y1))


def kernel(input, head_w, tail0_emb, tail0_proj, tail1_emb, tail1_proj):
    t1flat = tail1_emb.reshape(NT1 // 128, 128, D2).swapaxes(1, 2).reshape(-1)
    t0rm = with_layout_constraint(
        tail0_emb, Layout(major_to_minor=(0, 1), tiling=((8,),)))
    gh, g0, g1f = _stage_a(input, head_w, t0rm, t1flat)
    g1 = g1f.reshape(N_TOK, D2)
    ids2d = input.reshape(N_TOK, 1)
    p0t = tail0_proj.T
    p1t = tail1_proj.T
    out = pl.pallas_call(
        _stage_b_body,
        grid=(N_TOK // TB,),
        in_specs=[
            pl.BlockSpec((TB, 1), lambda i: (i, 0)),
            pl.BlockSpec((TB, NINP), lambda i: (i, 0)),
            pl.BlockSpec((TB, D1), lambda i: (i, 0)),
            pl.BlockSpec((TB, D2), lambda i: (i, 0)),
            pl.BlockSpec((D1, NINP), lambda i: (0, 0)),
            pl.BlockSpec((D2, NINP), lambda i: (0, 0)),
        ],
        out_specs=pl.BlockSpec((TB, NINP), lambda i: (i, 0)),
        out_shape=jax.ShapeDtypeStruct((N_TOK, NINP), jnp.float32),
    )(ids2d, gh, g0, g1, p0t, p1t)
    return out
